# baseline (speedup 1.0000x reference)
import functools
import math

import jax
import jax.numpy as jnp
from jax import lax
from jax.experimental import pallas as pl
from jax.experimental.pallas import tpu as pltpu
from jax.experimental.pallas import tpu_sc as plsc

EMB = 32
ATT = 16
DEG = 16
NSLOT = DEG + 1
NREL = 100
RPAD = 128
NODES = 200
SLOTS = NODES * NSLOT

_F32 = jnp.float32
_BF16 = jnp.bfloat16


def _hl(x):
    hi = x.astype(_BF16)
    lo = (x - hi.astype(_F32)).astype(_BF16)
    return hi, lo


def _dot_e(x, exact_bf16):
    xh, xl = _hl(x)
    return (jnp.dot(xh, exact_bf16, preferred_element_type=_F32)
            + jnp.dot(xl, exact_bf16, preferred_element_type=_F32))


def _dg3(a, b):
    ah, al = _hl(a)
    bh, bl = _hl(b)
    d = functools.partial(jnp.dot, preferred_element_type=_F32)
    return d(ah, bh) + d(ah, bl) + d(al, bh)


def _tc_stage(h_ref, msg_ref, mt_ref, wself_ref, khl_ref, curr_ref, kpn_ref):
    curr = _dg3(h_ref[...], wself_ref[...])
    e_all = jnp.concatenate([curr, msg_ref[...]], axis=0)
    mt = mt_ref[0]
    idx_k = jnp.concatenate(
        [jnp.full((NODES, 1), NREL, mt.dtype), mt], axis=0)
    lane_r = lax.broadcasted_iota(jnp.int32, (SLOTS, RPAD), 1)
    o_k = (idx_k == lane_r).astype(_BF16)
    e_rep = jnp.concatenate([e_all] * ATT, axis=1)
    f_sub = lax.broadcasted_iota(jnp.int32, (EMB * ATT, ATT), 0)
    f_lane = lax.broadcasted_iota(jnp.int32, (EMB * ATT, ATT), 1)
    fold_mat = ((f_sub // EMB) == f_lane).astype(_BF16)
    wg2 = jnp.dot(o_k, khl_ref[...], preferred_element_type=_F32)
    wg = wg2[:, :EMB * ATT] + wg2[:, EMB * ATT:]
    k_all = _dot_e(wg * e_rep, fold_mat)
    kpn_ref[...] = jnp.concatenate(
        [k_all[j * NODES:(j + 1) * NODES, :] for j in range(NSLOT)], axis=1)
    curr_ref[...] = curr


def _full16(x):
    return jnp.full((16,), x, jnp.int32)


def _splat(ref1d, off):
    return plsc.load_gather(ref1d, [_full16(off)])


def _make_sc_attend(bnum):
    info = plsc.get_sparse_core_info()
    nw = info.num_cores * info.num_subcores
    bpw = -(-((bnum + nw - 1) // nw) // 8) * 8
    ch = 8
    mesh = plsc.VectorSubcoreMesh(core_axis_name="c", subcore_axis_name="s")

    @functools.partial(
        pl.kernel, mesh=mesh,
        out_type=jax.ShapeDtypeStruct((bnum, ATT), _F32),
        compiler_params=pltpu.CompilerParams(needs_layout_passes=False),
        scratch_types=[
            pltpu.VMEM(((NREL + 1) * EMB * 2 * ATT,), _F32),
            pltpu.VMEM((ch * EMB,), _F32),
            pltpu.VMEM((ch * NSLOT * ATT,), _F32),
            pltpu.VMEM((ch * DEG * EMB,), _F32),
            pltpu.VMEM((ch * 32,), _F32),
            pltpu.VMEM((ch, ATT), _F32),
            pltpu.VMEM((DEG * ATT,), _F32),
            pltpu.VMEM((DEG * ATT,), _F32),
        ],
    )
    def attend(curr_hbm, kpn_hbm, msg_hbm, idx_hbm, qv_hbm,
               out_hbm, tbl_v, curr_v, kpn_v, msg_v, idx_v,
               out_v, qt_v, vv_v):
        wid = lax.axis_index("s") * info.num_cores + lax.axis_index("c")
        pltpu.sync_copy(qv_hbm, tbl_v)
        start = wid * bpw
        count = jnp.minimum(bpw, bnum - start)
        nch = (count + ch - 1) // ch
        iota = lax.iota(jnp.int32, 16)

        def project_slot(src_ref, ebase, rsplat):
            base = rsplat * (EMB * 2 * ATT) + iota
            q = jnp.zeros((16,), _F32)
            v = jnp.zeros((16,), _F32)
            for e in range(EMB):
                ev = _splat(src_ref, ebase + e)
                rq = plsc.load_gather(tbl_v, [base + e * (2 * ATT)])
                rv = plsc.load_gather(tbl_v, [base + e * (2 * ATT) + ATT])
                q = q + ev * rq
                v = v + ev * rv
            return q, v

        def node_body(ln, _):
            r0 = _splat(idx_v, ln * 32).astype(jnp.int32)
            q_self, v_self = project_slot(curr_v, ln * EMB, r0)
            for j in range(1, NSLOT):
                rj = _splat(idx_v, ln * 32 + j).astype(jnp.int32)
                qj, vj = project_slot(msg_v, (ln * DEG + (j - 1)) * EMB, rj)
                plsc.store_scatter(qt_v, [iota * DEG + (j - 1)], qj)
                vv_v[pl.ds((j - 1) * ATT, ATT)] = vj
            pooled = jnp.zeros((16,), _F32)
            inv = 1.0 / math.sqrt(ATT)
            for j in range(NSLOT):
                kv = kpn_v[pl.ds(ln * (NSLOT * ATT) + j * ATT, ATT)]
                s_m = jnp.zeros((16,), _F32)
                for a in range(ATT):
                    qa = qt_v[pl.ds(a * DEG, DEG)]
                    ka = _splat(kpn_v, ln * (NSLOT * ATT) + j * ATT + a)
                    s_m = s_m + qa * ka
                s_m = s_m * inv
                s0 = jnp.sum(q_self * kv) * inv
                m = jnp.maximum(jnp.max(s_m), s0)
                ex = jnp.exp(s_m - m)
                ex0v = jnp.exp(jnp.full((16,), s0 - m, _F32))
                ex0 = jnp.max(ex0v)
                den = jnp.full((16,), jnp.sum(ex) + ex0, _F32)
                wv = ex0v / den
                vj = v_self if j == 0 else vv_v[pl.ds((j - 1) * ATT, ATT)]
                pooled = pooled + wv * vj
            out_v[ln, pl.ds(0, 16)] = pooled
            return 0

        def chunk_body(c, _):
            n0 = start + jnp.minimum(c * ch, count - ch)
            pltpu.sync_copy(curr_hbm.at[pl.ds(n0 * EMB, ch * EMB)], curr_v)
            pltpu.sync_copy(
                kpn_hbm.at[pl.ds(n0 * NSLOT * ATT, ch * NSLOT * ATT)], kpn_v)
            pltpu.sync_copy(
                msg_hbm.at[pl.ds(n0 * DEG * EMB, ch * DEG * EMB)], msg_v)
            pltpu.sync_copy(idx_hbm.at[pl.ds(n0 * 32, ch * 32)], idx_v)
            lax.fori_loop(0, ch, node_body, 0)
            pltpu.sync_copy(out_v, out_hbm.at[pl.ds(n0, ch)])
            return 0

        lax.fori_loop(0, nch, chunk_body, 0)

    return attend


def kernel(h, msg, r_label, msg_type, msg_r_label, self_loop_weight,
           relational_Q, relational_K, relational_V, ffn_w, ffn_b):
    bnum = h.shape[0]
    nblk = bnum // NODES
    inp = h.shape[1]

    msg_jm = (msg.reshape(nblk, NODES, DEG, EMB).transpose(0, 2, 1, 3)
              .reshape(bnum * DEG, EMB))
    mt3 = (msg_type.astype(jnp.int32).reshape(nblk, NODES, DEG)
           .transpose(0, 2, 1).reshape(nblk, NODES * DEG, 1))

    kflat = relational_K.transpose(0, 2, 1).reshape(NREL + 1, EMB * ATT)
    kflat = jnp.concatenate(
        [kflat, jnp.zeros((RPAD - (NREL + 1), EMB * ATT), kflat.dtype)], axis=0)
    kh, kl = _hl(kflat)
    khl = jnp.concatenate([kh, kl], axis=1)

    full = lambda shape: pl.BlockSpec(shape, lambda i: (0,) * len(shape))
    curr, kpn = pl.pallas_call(
        _tc_stage,
        grid=(nblk,),
        in_specs=[
            pl.BlockSpec((NODES, inp), lambda i: (i, 0)),
            pl.BlockSpec((NODES * DEG, EMB), lambda i: (i, 0)),
            pl.BlockSpec((1, NODES * DEG, 1), lambda i: (i, 0, 0)),
            full((inp, EMB)),
            full((RPAD, 2 * EMB * ATT)),
        ],
        out_specs=[pl.BlockSpec((NODES, EMB), lambda i: (i, 0)),
                   pl.BlockSpec((NODES, NSLOT * ATT), lambda i: (i, 0))],
        out_shape=[jax.ShapeDtypeStruct((bnum, EMB), _F32),
                   jax.ShapeDtypeStruct((bnum, NSLOT * ATT), _F32)],
    )(h, msg_jm, mt3, self_loop_weight, khl)

    qv = jnp.concatenate([relational_Q, relational_V], axis=2).reshape(-1)
    idxpk = jnp.concatenate(
        [r_label.astype(_F32)[:, None], msg_r_label.astype(_F32),
         jnp.zeros((bnum, 32 - NSLOT), _F32)], axis=1)
    msg2d = msg.reshape(bnum * DEG, EMB)

    attend = _make_sc_attend(bnum)
    pooled = attend(curr.reshape(-1), kpn.reshape(-1), msg2d.reshape(-1),
                    idxpk.reshape(-1), qv)

    def _ffn(p_ref, w_ref, b_ref, o_ref):
        o_ref[...] = _dg3(p_ref[...], w_ref[...]) + b_ref[...]

    full2 = lambda shape: pl.BlockSpec(shape, lambda i: (0,) * len(shape))
    out = pl.pallas_call(
        _ffn, grid=(10,),
        in_specs=[pl.BlockSpec((bnum // 10, ATT), lambda i: (i, 0)),
                  full2((ATT, EMB)), full2((1, EMB))],
        out_specs=pl.BlockSpec((bnum // 10, EMB), lambda i: (i, 0)),
        out_shape=jax.ShapeDtypeStruct((bnum, EMB), _F32),
    )(pooled, ffn_w.T, ffn_b.reshape(1, EMB))
    return out


# speedup vs baseline: 1.4273x; 1.4273x over previous
import functools
import math

import jax
import jax.numpy as jnp
from jax import lax
from jax.experimental import pallas as pl
from jax.experimental.pallas import tpu as pltpu
from jax.experimental.pallas import tpu_sc as plsc

EMB = 32
ATT = 16
DEG = 16
NSLOT = DEG + 1
NREL = 100
RPAD = 128
NODES = 200
SLOTS = NODES * NSLOT

_F32 = jnp.float32
_BF16 = jnp.bfloat16


def _hl(x):
    hi = x.astype(_BF16)
    lo = (x - hi.astype(_F32)).astype(_BF16)
    return hi, lo


def _dot_e(x, exact_bf16):
    xh, xl = _hl(x)
    return (jnp.dot(xh, exact_bf16, preferred_element_type=_F32)
            + jnp.dot(xl, exact_bf16, preferred_element_type=_F32))


def _dg3(a, b):
    ah, al = _hl(a)
    bh, bl = _hl(b)
    d = functools.partial(jnp.dot, preferred_element_type=_F32)
    return d(ah, bh) + d(ah, bl) + d(al, bh)


def _tc_stage(h_ref, msg_ref, mt_ref, wself_ref, khl_ref, curr_ref, kpn_ref):
    curr = _dg3(h_ref[...], wself_ref[...])
    e_all = jnp.concatenate([curr, msg_ref[...]], axis=0)
    mt = mt_ref[0]
    idx_k = jnp.concatenate(
        [jnp.full((NODES, 1), NREL, mt.dtype), mt], axis=0)
    lane_r = lax.broadcasted_iota(jnp.int32, (SLOTS, RPAD), 1)
    o_k = (idx_k == lane_r).astype(_BF16)
    e_rep = jnp.concatenate([e_all] * ATT, axis=1)
    f_sub = lax.broadcasted_iota(jnp.int32, (EMB * ATT, ATT), 0)
    f_lane = lax.broadcasted_iota(jnp.int32, (EMB * ATT, ATT), 1)
    fold_mat = ((f_sub // EMB) == f_lane).astype(_BF16)
    wg2 = jnp.dot(o_k, khl_ref[...], preferred_element_type=_F32)
    wg = wg2[:, :EMB * ATT] + wg2[:, EMB * ATT:]
    k_all = _dot_e(wg * e_rep, fold_mat)
    kpn_ref[...] = jnp.concatenate(
        [k_all[j * NODES:(j + 1) * NODES, :] for j in range(NSLOT)], axis=1)
    curr_ref[...] = curr


def _full16(x):
    return jnp.full((16,), x, jnp.int32)


def _splat(ref1d, off):
    return plsc.load_gather(ref1d, [_full16(off)])


def _make_sc_attend(bnum):
    info = plsc.get_sparse_core_info()
    nw = info.num_cores * info.num_subcores
    bpw = -(-((bnum + nw - 1) // nw) // 8) * 8
    ch = 16
    mesh = plsc.VectorSubcoreMesh(core_axis_name="c", subcore_axis_name="s")

    @functools.partial(
        pl.kernel, mesh=mesh,
        out_type=jax.ShapeDtypeStruct((bnum, ATT), _F32),
        compiler_params=pltpu.CompilerParams(needs_layout_passes=False),
        scratch_types=[
            pltpu.VMEM(((NREL + 1) * EMB * 2 * ATT,), _F32),
            pltpu.VMEM((ch * EMB,), _F32),
            pltpu.VMEM((ch * NSLOT * ATT,), _F32),
            pltpu.VMEM((ch * DEG * EMB,), _F32),
            pltpu.VMEM((ch * 32,), _F32),
            pltpu.VMEM((ch, ATT), _F32),
            pltpu.VMEM((DEG * ATT,), _F32),
            pltpu.VMEM((DEG * ATT,), _F32),
        ],
    )
    def attend(curr_hbm, kpn_hbm, msg_hbm, idx_hbm, qv_hbm,
               out_hbm, tbl_v, curr_v, kpn_v, msg_v, idx_v,
               out_v, qt_v, vv_v):
        wid = lax.axis_index("s") * info.num_cores + lax.axis_index("c")
        pltpu.sync_copy(qv_hbm, tbl_v)
        start = wid * bpw
        count = jnp.minimum(bpw, bnum - start)
        nch = (count + ch - 1) // ch
        iota = lax.iota(jnp.int32, 16)

        def project_slot(src_ref, ebase, rsplat):
            base = rsplat * (EMB * 2 * ATT) + iota
            e_lo = src_ref[pl.ds(ebase, 16)]
            e_hi = src_ref[pl.ds(ebase + 16, 16)]
            q = jnp.zeros((16,), _F32)
            v = jnp.zeros((16,), _F32)
            for e in range(EMB):
                src = e_lo if e < 16 else e_hi
                ev = src.at[_full16(e % 16)].get(mode="promise_in_bounds")
                rq = plsc.load_gather(tbl_v, [base + e * (2 * ATT)])
                rv = plsc.load_gather(tbl_v, [base + e * (2 * ATT) + ATT])
                q = q + ev * rq
                v = v + ev * rv
            return q, v

        def node_body(ln, _):
            r0 = _splat(idx_v, ln * 32).astype(jnp.int32)
            q_self, v_self = project_slot(curr_v, ln * EMB, r0)
            for j in range(1, NSLOT):
                rj = _splat(idx_v, ln * 32 + j).astype(jnp.int32)
                qj, vj = project_slot(msg_v, (ln * DEG + (j - 1)) * EMB, rj)
                plsc.store_scatter(qt_v, [iota * DEG + (j - 1)], qj)
                vv_v[pl.ds((j - 1) * ATT, ATT)] = vj
            pooled = jnp.zeros((16,), _F32)
            inv = 1.0 / math.sqrt(ATT)
            for j in range(NSLOT):
                kv = kpn_v[pl.ds(ln * (NSLOT * ATT) + j * ATT, ATT)]
                s_m = jnp.zeros((16,), _F32)
                for a in range(ATT):
                    qa = qt_v[pl.ds(a * DEG, DEG)]
                    ka = _splat(kpn_v, ln * (NSLOT * ATT) + j * ATT + a)
                    s_m = s_m + qa * ka
                s_m = s_m * inv
                s0 = jnp.sum(q_self * kv) * inv
                m = jnp.maximum(jnp.max(s_m), s0)
                ex = jnp.exp(s_m - m)
                ex0v = jnp.exp(jnp.full((16,), s0 - m, _F32))
                ex0 = jnp.max(ex0v)
                den = jnp.full((16,), jnp.sum(ex) + ex0, _F32)
                wv = ex0v / den
                vj = v_self if j == 0 else vv_v[pl.ds((j - 1) * ATT, ATT)]
                pooled = pooled + wv * vj
            out_v[ln, pl.ds(0, 16)] = pooled
            return 0

        def chunk_body(c, _):
            n0 = start + jnp.minimum(c * ch, count - ch)
            pltpu.sync_copy(curr_hbm.at[pl.ds(n0 * EMB, ch * EMB)], curr_v)
            pltpu.sync_copy(
                kpn_hbm.at[pl.ds(n0 * NSLOT * ATT, ch * NSLOT * ATT)], kpn_v)
            pltpu.sync_copy(
                msg_hbm.at[pl.ds(n0 * DEG * EMB, ch * DEG * EMB)], msg_v)
            pltpu.sync_copy(idx_hbm.at[pl.ds(n0 * 32, ch * 32)], idx_v)
            lax.fori_loop(0, ch, node_body, 0)
            pltpu.sync_copy(out_v, out_hbm.at[pl.ds(n0, ch)])
            return 0

        lax.fori_loop(0, nch, chunk_body, 0)

    return attend


def kernel(h, msg, r_label, msg_type, msg_r_label, self_loop_weight,
           relational_Q, relational_K, relational_V, ffn_w, ffn_b):
    bnum = h.shape[0]
    nblk = bnum // NODES
    inp = h.shape[1]

    msg_jm = (msg.reshape(nblk, NODES, DEG, EMB).transpose(0, 2, 1, 3)
              .reshape(bnum * DEG, EMB))
    mt3 = (msg_type.astype(jnp.int32).reshape(nblk, NODES, DEG)
           .transpose(0, 2, 1).reshape(nblk, NODES * DEG, 1))

    kflat = relational_K.transpose(0, 2, 1).reshape(NREL + 1, EMB * ATT)
    kflat = jnp.concatenate(
        [kflat, jnp.zeros((RPAD - (NREL + 1), EMB * ATT), kflat.dtype)], axis=0)
    kh, kl = _hl(kflat)
    khl = jnp.concatenate([kh, kl], axis=1)

    full = lambda shape: pl.BlockSpec(shape, lambda i: (0,) * len(shape))
    curr, kpn = pl.pallas_call(
        _tc_stage,
        grid=(nblk,),
        in_specs=[
            pl.BlockSpec((NODES, inp), lambda i: (i, 0)),
            pl.BlockSpec((NODES * DEG, EMB), lambda i: (i, 0)),
            pl.BlockSpec((1, NODES * DEG, 1), lambda i: (i, 0, 0)),
            full((inp, EMB)),
            full((RPAD, 2 * EMB * ATT)),
        ],
        out_specs=[pl.BlockSpec((NODES, EMB), lambda i: (i, 0)),
                   pl.BlockSpec((NODES, NSLOT * ATT), lambda i: (i, 0))],
        out_shape=[jax.ShapeDtypeStruct((bnum, EMB), _F32),
                   jax.ShapeDtypeStruct((bnum, NSLOT * ATT), _F32)],
    )(h, msg_jm, mt3, self_loop_weight, khl)

    qv = jnp.concatenate([relational_Q, relational_V], axis=2).reshape(-1)
    idxpk = jnp.concatenate(
        [r_label.astype(_F32)[:, None], msg_r_label.astype(_F32),
         jnp.zeros((bnum, 32 - NSLOT), _F32)], axis=1)
    msg2d = msg.reshape(bnum * DEG, EMB)

    attend = _make_sc_attend(bnum)
    pooled = attend(curr.reshape(-1), kpn.reshape(-1), msg2d.reshape(-1),
                    idxpk.reshape(-1), qv)

    def _ffn(p_ref, w_ref, b_ref, o_ref):
        o_ref[...] = _dg3(p_ref[...], w_ref[...]) + b_ref[...]

    full2 = lambda shape: pl.BlockSpec(shape, lambda i: (0,) * len(shape))
    out = pl.pallas_call(
        _ffn, grid=(10,),
        in_specs=[pl.BlockSpec((bnum // 10, ATT), lambda i: (i, 0)),
                  full2((ATT, EMB)), full2((1, EMB))],
        out_specs=pl.BlockSpec((bnum // 10, EMB), lambda i: (i, 0)),
        out_shape=jax.ShapeDtypeStruct((bnum, EMB), _F32),
    )(pooled, ffn_w.T, ffn_b.reshape(1, EMB))
    return out
